# baseline (speedup 1.0000x reference)
import jax
import jax.numpy as jnp
from jax.experimental import pallas as pl
from jax.experimental.pallas import tpu as pltpu


def _cheb(x, edge_index, edge_weight, W, b):
    n = x.shape[0]
    row = edge_index[0]
    col = edge_index[1]
    deg = jnp.zeros((n,), x.dtype).at[row].add(edge_weight)
    safe = jnp.where(deg > 0, deg, 1.0)
    dis = jnp.where(deg > 0, 1.0 / jnp.sqrt(safe), 0.0)
    norm = -dis[row] * edge_weight * dis[col]

    def prop(h):
        return jnp.zeros_like(h).at[col].add(norm[:, None] * h[row])

    Tx0 = x
    out = Tx0 @ W[0]
    Tx1 = prop(Tx0)
    out = out + Tx1 @ W[1]
    for k in range(2, W.shape[0]):
        Tx2 = 2.0 * prop(Tx1) - Tx0
        out = out + Tx2 @ W[k]
        Tx0, Tx1 = Tx1, Tx2
    return out + b


def _fc_body(h_ref, w1_ref, b1_ref, w2_ref, b2_ref, w3_ref, b3_ref, o_ref):
    h1 = jnp.dot(h_ref[...], w1_ref[...],
                 preferred_element_type=jnp.float32) + b1_ref[...]
    h2 = jnp.dot(h1, w2_ref[...], preferred_element_type=jnp.float32) + b2_ref[...]
    h3 = jnp.dot(h2, w3_ref[...], preferred_element_type=jnp.float32) + b3_ref[...]
    o_ref[...] = h3


def _fc_head(hflat, fcW1, fcb1, fcW2, fcb2, fcW3, fcb3):
    return pl.pallas_call(
        _fc_body,
        out_shape=jax.ShapeDtypeStruct((1, 2), jnp.float32),
    )(hflat, fcW1, fcb1, fcW2, fcb2, fcW3, fcb3)


def kernel(x, edge_index, edge_weight, W1, b1, W2, b2, W3, b3, W4, b4, W5, b5,
           fcW1, fcb1, fcW2, fcb2, fcW3, fcb3):
    h = x
    for (W, b) in [(W1, b1), (W2, b2), (W3, b3), (W4, b4), (W5, b5)]:
        h = jax.nn.relu(_cheb(h, edge_index, edge_weight, W, b))
    hflat = h.reshape(1, -1)
    return _fc_head(hflat, fcW1, fcb1, fcW2, fcb2, fcW3, fcb3)


# speedup vs baseline: 1.9047x; 1.9047x over previous
import functools

import jax
import jax.numpy as jnp
from jax import lax
from jax.experimental import pallas as pl
from jax.experimental.pallas import tpu as pltpu
from jax.experimental.pallas import tpu_sc as plsc

NC = 2
NS = 16
NW = NC * NS
L = 16
B = 128

_MESH = functools.partial(
    plsc.VectorSubcoreMesh, core_axis_name="c", subcore_axis_name="s")
_SC_PARAMS = pltpu.CompilerParams(needs_layout_passes=False)


def _wid():
    return lax.axis_index("s") * NC + lax.axis_index("c")


def _pad_to(a, m, value):
    r = (-a.shape[0]) % m
    if r == 0:
        return a
    return jnp.concatenate([a, jnp.full((r,), value, a.dtype)])


def _deg_body(row_hbm, w_hbm, out_hbm, deg_v, idx_v, wv_v, sem):
    E2 = row_hbm.shape[0]
    EPT = E2 // NW
    npad = deg_v.shape[0]
    wid = _wid()

    def zero(i, _):
        deg_v[pl.ds(i * L, L)] = jnp.zeros((L,), jnp.float32)
        return 0
    lax.fori_loop(0, npad // L, zero, 0)

    base0 = wid * EPT

    def chunk(ci, _):
        base = base0 + ci * B
        pltpu.sync_copy(row_hbm.at[pl.ds(base, B)], idx_v)
        pltpu.sync_copy(w_hbm.at[pl.ds(base, B)], wv_v)
        for g in range(B // L):
            r = idx_v[pl.ds(g * L, L)]
            w = wv_v[pl.ds(g * L, L)]
            plsc.addupdate_scatter(deg_v, [r], w)
        return 0
    lax.fori_loop(0, EPT // B, chunk, 0)
    pltpu.sync_copy(deg_v, out_hbm.at[pl.ds(wid * npad, npad)])


def _deg_partials(srt_row2, srt_w2, npad):
    E2 = srt_row2.shape[0]
    assert (E2 // NW) % B == 0 and npad % B == 0
    k = pl.kernel(
        _deg_body,
        out_type=jax.ShapeDtypeStruct((NW * npad,), jnp.float32),
        mesh=_MESH(),
        compiler_params=_SC_PARAMS,
        scratch_types=[
            pltpu.VMEM((npad,), jnp.float32),
            pltpu.VMEM((B,), jnp.int32),
            pltpu.VMEM((B,), jnp.float32),
            pltpu.SemaphoreType.DMA,
        ],
    )
    return k(srt_row2, srt_w2).reshape(NW, npad)


def _dis_body(parts_ref, o_ref):
    deg = jnp.sum(parts_ref[...], axis=0, keepdims=True)
    safe = jnp.where(deg > 0, deg, 1.0)
    o_ref[...] = jnp.where(deg > 0, 1.0 / jnp.sqrt(safe), 0.0)


def _dis(parts):
    npad = parts.shape[1]
    return pl.pallas_call(
        _dis_body,
        out_shape=jax.ShapeDtypeStruct((1, npad), jnp.float32),
    )(parts)


def _norm_body(row_hbm, col_hbm, w_hbm, dis_hbm, out_hbm,
               dis_v, r_v, c_v, w_v, o_v, sem):
    E2 = row_hbm.shape[0]
    EPT = E2 // NW
    wid = _wid()
    pltpu.sync_copy(dis_hbm, dis_v)
    base0 = wid * EPT

    def chunk(ci, _):
        base = base0 + ci * B
        pltpu.sync_copy(row_hbm.at[pl.ds(base, B)], r_v)
        pltpu.sync_copy(col_hbm.at[pl.ds(base, B)], c_v)
        pltpu.sync_copy(w_hbm.at[pl.ds(base, B)], w_v)
        for g in range(B // L):
            r = r_v[pl.ds(g * L, L)]
            c = c_v[pl.ds(g * L, L)]
            w = w_v[pl.ds(g * L, L)]
            dr = plsc.load_gather(dis_v, [r])
            dc = plsc.load_gather(dis_v, [c])
            o_v[pl.ds(g * L, L)] = -(dr * w * dc)
        pltpu.sync_copy(o_v, out_hbm.at[pl.ds(base, B)])
        return 0
    lax.fori_loop(0, EPT // B, chunk, 0)


def _norm(srt_row2, srt_col2, srt_w2, dis_pad):
    E2 = srt_row2.shape[0]
    npad = dis_pad.shape[0]
    k = pl.kernel(
        _norm_body,
        out_type=jax.ShapeDtypeStruct((E2,), jnp.float32),
        mesh=_MESH(),
        compiler_params=_SC_PARAMS,
        scratch_types=[
            pltpu.VMEM((npad,), jnp.float32),
            pltpu.VMEM((B,), jnp.int32),
            pltpu.VMEM((B,), jnp.int32),
            pltpu.VMEM((B,), jnp.float32),
            pltpu.VMEM((B,), jnp.float32),
            pltpu.SemaphoreType.DMA,
        ],
    )
    return k(srt_row2, srt_col2, srt_w2, dis_pad)


NPT = 320


def _prop_body(h_hbm, row_hbm, col_hbm, nrm_hbm, bnd_hbm, out_hbm,
               acc_v, rows_v, idx_v, ib_v, nm_v, bnd_v, sem):
    C = acc_v.shape[1]
    wid = _wid()
    nlo = wid * NPT

    pltpu.sync_copy(bnd_hbm, bnd_v)
    iota = lax.iota(jnp.int32, L)

    def bget(q):
        sel = jnp.where(iota == q, bnd_v[pl.ds(0, L)], 0)
        sel = sel + jnp.where(iota + 16 == q, bnd_v[pl.ds(16, L)], 0)
        sel = sel + jnp.where(iota + 32 == q, bnd_v[pl.ds(32, L)], 0)
        return jnp.max(sel)

    s_w = bget(wid)
    e_w = bget(wid + 1)

    def zero(i, _):
        for j in range(C // L):
            acc_v[i, pl.ds(j * L, L)] = jnp.zeros((L,), jnp.float32)
        return 0
    lax.fori_loop(0, NPT, zero, 0)

    c0 = s_w // B
    c1 = (e_w + B - 1) // B

    def chunk(ci, _):
        base = ci * B
        pltpu.sync_copy(row_hbm.at[pl.ds(base, B)], idx_v)
        pltpu.sync_copy(nrm_hbm.at[pl.ds(base, B)], nm_v)
        pltpu.async_copy(h_hbm.at[idx_v], rows_v, sem).wait()
        pltpu.sync_copy(col_hbm.at[pl.ds(base, B)], ib_v)
        for g in range(B // L):
            eg = base + g * L + iota
            m = (eg >= s_w) & (eg < e_w)
            nm16 = jnp.where(m, nm_v[pl.ds(g * L, L)], 0.0)
            cl16 = jnp.clip(ib_v[pl.ds(g * L, L)] - nlo, 0, NPT - 1)
            for lane in range(L):
                e = g * L + lane
                sel = iota == lane
                ns = jnp.sum(jnp.where(sel, nm16, 0.0))
                rl = jnp.sum(jnp.where(sel, cl16, 0))
                ridx = jnp.zeros((L,), jnp.int32) + rl
                for j in range(C // L):
                    v = rows_v[e, pl.ds(j * L, L)] * ns
                    plsc.addupdate_scatter(
                        acc_v,
                        [ridx, jnp.full((L,), j * L, jnp.int32) + iota], v)
        return 0
    lax.fori_loop(c0, c1, chunk, 0)
    pltpu.sync_copy(acc_v, out_hbm.at[pl.ds(nlo, NPT)])


def _make_prop(C):
    return pl.kernel(
        _prop_body,
        out_type=jax.ShapeDtypeStruct((NW * NPT, C), jnp.float32),
        mesh=_MESH(),
        compiler_params=_SC_PARAMS,
        scratch_types=[
            pltpu.VMEM((NPT, C), jnp.float32),
            pltpu.VMEM((B, C), jnp.float32),
            pltpu.VMEM((B,), jnp.int32),
            pltpu.VMEM((B,), jnp.int32),
            pltpu.VMEM((B,), jnp.float32),
            pltpu.VMEM((48,), jnp.int32),
            pltpu.SemaphoreType.DMA,
        ],
    )


def _cheb_mm_body(h_ref, p0_ref, p1_ref, w_ref, b_ref, o_ref):
    h = h_ref[...]
    p0 = p0_ref[...]
    p1 = p1_ref[...]
    acc = jnp.dot(h, w_ref[0], preferred_element_type=jnp.float32)
    acc += jnp.dot(p0, w_ref[1], preferred_element_type=jnp.float32)
    acc += jnp.dot(2.0 * p1 - h, w_ref[2], preferred_element_type=jnp.float32)
    o_ref[...] = jnp.maximum(acc + b_ref[...], 0.0)


def _cheb_mm(h, p0, p1, W, b):
    n = h.shape[0]
    di = h.shape[1]
    do = W.shape[2]
    BN = 2000
    grid = (n + BN - 1) // BN
    return pl.pallas_call(
        _cheb_mm_body,
        grid=(grid,),
        in_specs=[
            pl.BlockSpec((BN, di), lambda i: (i, 0)),
            pl.BlockSpec((BN, di), lambda i: (i, 0)),
            pl.BlockSpec((BN, di), lambda i: (i, 0)),
            pl.BlockSpec((3, di, do), lambda i: (0, 0, 0)),
            pl.BlockSpec((1, do), lambda i: (0, 0)),
        ],
        out_specs=pl.BlockSpec((BN, do), lambda i: (i, 0)),
        out_shape=jax.ShapeDtypeStruct((n, do), jnp.float32),
    )(h, p0, p1, W, b.reshape(1, do))


def _fc_body(h_ref, w1_ref, b1_ref, w2_ref, b2_ref, w3_ref, b3_ref, o_ref):
    h1 = jnp.dot(h_ref[...], w1_ref[...],
                 preferred_element_type=jnp.float32) + b1_ref[...]
    h2 = jnp.dot(h1, w2_ref[...], preferred_element_type=jnp.float32) + b2_ref[...]
    h3 = jnp.dot(h2, w3_ref[...], preferred_element_type=jnp.float32) + b3_ref[...]
    o_ref[...] = h3


def _fc_head(hflat, fcW1, fcb1, fcW2, fcb2, fcW3, fcb3):
    return pl.pallas_call(
        _fc_body,
        out_shape=jax.ShapeDtypeStruct((1, 2), jnp.float32),
    )(hflat, fcW1, fcb1, fcW2, fcb2, fcW3, fcb3)


def kernel(x, edge_index, edge_weight, W1, b1, W2, b2, W3, b3, W4, b4, W5, b5,
           fcW1, fcb1, fcW2, fcb2, fcW3, fcb3):
    n, T = x.shape
    E = edge_index.shape[1]
    assert NW * NPT >= n

    row = edge_index[0]
    col = edge_index[1]
    srt_col, srt_row, srt_w = lax.sort((col, row, edge_weight), num_keys=1)
    node_edges = jnp.searchsorted(
        srt_col, jnp.arange(NW + 1, dtype=jnp.int32) * NPT).astype(jnp.int32)
    bounds = jnp.zeros((48,), jnp.int32).at[: NW + 1].set(node_edges)

    srt_row2 = _pad_to(srt_row, NW * B, 0)
    srt_col2 = _pad_to(srt_col, NW * B, 0)
    srt_w2 = _pad_to(srt_w, NW * B, 0.0)

    npad = ((n + B - 1) // B) * B
    parts = _deg_partials(srt_row2, srt_w2, npad)
    dis_pad = _dis(parts)[0]
    srt_norm2 = _norm(srt_row2, srt_col2, srt_w2, dis_pad)

    prop_k = _make_prop(T)

    def prop(h):
        return prop_k(h, srt_row2, srt_col2, srt_norm2, bounds)[:n]

    h = x
    for (W, b) in [(W1, b1), (W2, b2), (W3, b3), (W4, b4), (W5, b5)]:
        p0 = prop(h)
        p1 = prop(p0)
        h = _cheb_mm(h, p0, p1, W, b)

    hflat = h.reshape(1, -1)
    return _fc_head(hflat, fcW1, fcb1, fcW2, fcb2, fcW3, fcb3)
